# baseline (speedup 1.0000x reference)
import functools

import jax
import jax.numpy as jnp
from jax import lax
from jax.experimental import pallas as pl
from jax.experimental.pallas import tpu as pltpu
from jax.experimental.pallas import tpu_sc as plsc

B, L, CL = 1024, 50, 16
WORD_DIM, CHAR_DIM, HIDDEN = 128, 64, 128
CHAR_VOCAB = 1000
T = B * L
NC, NS = 2, 16
NW = NC * NS
TPW = T // NW
WCH = 80
NWCH = TPW // WCH
CT = 8
NCT = TPW // CT
NCIR = TPW * CL // 128
CNB = 4

_sc_mesh = plsc.VectorSubcoreMesh(core_axis_name="c", subcore_axis_name="s")


@functools.partial(
    pl.kernel,
    mesh=_sc_mesh,
    out_type=(
        jax.ShapeDtypeStruct((T, WORD_DIM), jnp.float32),
        jax.ShapeDtypeStruct((T, HIDDEN), jnp.float32),
    ),
    scratch_types=[
        pltpu.VMEM((NWCH, WCH), jnp.int32),
        pltpu.VMEM((NCIR, 128), jnp.int32),
        pltpu.VMEM((2, WCH, WORD_DIM), jnp.float32),
        pltpu.VMEM((CNB, CT * CL, HIDDEN), jnp.float32),
        pltpu.VMEM((2, CT, HIDDEN), jnp.float32),
        [pltpu.SemaphoreType.DMA] * CNB,
        [pltpu.SemaphoreType.DMA] * 2,
    ],
)
def _sc_gather(wt_hbm, cp_hbm, widx_hbm, cidx_hbm, wout_hbm, cout_hbm,
               widx_v, cidx_v, wrows_v, crows_v, csum_v, gsems, osems):
    wid = lax.axis_index("s") * NC + lax.axis_index("c")
    pltpu.sync_copy(widx_hbm.at[wid], widx_v)
    pltpu.sync_copy(cidx_hbm.at[wid], cidx_v)
    base = wid * TPW

    pltpu.async_copy(wt_hbm.at[widx_v.at[0]], wrows_v.at[0], gsems[0])

    def wpair(p, _):
        for b in range(2):
            j = p * 2 + b

            @pl.when(j + 1 < NWCH)
            def _():
                @pl.when(j >= 1)
                def _():
                    pltpu.make_async_copy(
                        wrows_v.at[1 - b],
                        wout_hbm.at[pl.ds(base + (j - 1) * WCH, WCH)],
                        osems[1 - b]).wait()
                pltpu.async_copy(wt_hbm.at[widx_v.at[j + 1]],
                                 wrows_v.at[1 - b], gsems[1 - b])

            pltpu.make_async_copy(wt_hbm.at[widx_v.at[j]],
                                  wrows_v.at[b], gsems[b]).wait()
            pltpu.async_copy(wrows_v.at[b],
                             wout_hbm.at[pl.ds(base + j * WCH, WCH)],
                             osems[b])
        return 0

    lax.fori_loop(0, NWCH // 2, wpair, 0)
    for b in range(2):
        pltpu.make_async_copy(
            wrows_v.at[b],
            wout_hbm.at[pl.ds(base + (NWCH - 2 + b) * WCH, WCH)],
            osems[b]).wait()

    for s in range(CNB - 1):
        pltpu.async_copy(cp_hbm.at[cidx_v.at[s]], crows_v.at[s], gsems[s])

    def cgroup(p, _):
        for b in range(CNB):
            j = p * CNB + b

            nslot = (b + CNB - 1) % CNB

            @pl.when(j + CNB - 1 < NCT)
            def _():
                pltpu.async_copy(cp_hbm.at[cidx_v.at[j + CNB - 1]],
                                 crows_v.at[nslot], gsems[nslot])

            pltpu.make_async_copy(cp_hbm.at[cidx_v.at[j]],
                                  crows_v.at[b], gsems[b]).wait()
            o = b % 2

            @pl.when(j >= 2)
            def _():
                pltpu.make_async_copy(
                    csum_v.at[o],
                    cout_hbm.at[pl.ds(base + (j - 2) * CT, CT)],
                    osems[o]).wait()

            def tok(i, _):
                for r in range(HIDDEN // 16):
                    sl = pl.ds(r * 16, 16)
                    acc = crows_v[b, i * CL, sl]
                    for k in range(1, CL):
                        acc = acc + crows_v[b, i * CL + k, sl]
                    csum_v[o, i, sl] = acc
                return 0

            lax.fori_loop(0, CT, tok, 0)
            pltpu.async_copy(csum_v.at[o],
                             cout_hbm.at[pl.ds(base + j * CT, CT)],
                             osems[o])
        return 0

    lax.fori_loop(0, NCT // CNB, cgroup, 0)
    for o in range(2):
        pltpu.make_async_copy(
            csum_v.at[o],
            cout_hbm.at[pl.ds(base + (NCT - 2 + o) * CT, CT)],
            osems[o]).wait()


def _cp_body(ct_ref, wpc_ref, cp_ref):
    ct = ct_ref[...]
    for k in range(CL):
        cp_ref[pl.ds(k * CHAR_VOCAB, CHAR_VOCAB), :] = jnp.dot(
            ct, wpc_ref[k], preferred_element_type=jnp.float32)


_cp_call = pl.pallas_call(
    _cp_body,
    in_specs=[
        pl.BlockSpec((CHAR_VOCAB, CHAR_DIM), lambda: (0, 0)),
        pl.BlockSpec((CL, CHAR_DIM, HIDDEN), lambda: (0, 0, 0)),
    ],
    out_specs=pl.BlockSpec((CL * CHAR_VOCAB, HIDDEN), lambda: (0, 0)),
    out_shape=jax.ShapeDtypeStruct((CL * CHAR_VOCAB, HIDDEN), jnp.float32),
)


TB = 2048
GRID = T // TB


def _tc_body(wd, cs, wpwT, wg0T, bg0, wt0T, bt0, wg1T, bg1, wt1T, bt1, out):
    x = jnp.dot(wd[...], wpwT[...], preferred_element_type=jnp.float32)
    x += cs[...]
    for wgT, bg, wtT, bt in ((wg0T, bg0, wt0T, bt0), (wg1T, bg1, wt1T, bt1)):
        zg = jnp.dot(x, wgT[...], preferred_element_type=jnp.float32) + bg[...]
        g = 1.0 / (1.0 + jnp.exp(-zg))
        zt = jnp.dot(x, wtT[...], preferred_element_type=jnp.float32) + bt[...]
        x = g * jnp.maximum(zt, 0.0) + (1.0 - g) * x
    out[...] = x


def _full(shape):
    return pl.BlockSpec(shape, lambda i: (0, 0))


_tc_call = pl.pallas_call(
    _tc_body,
    grid=(GRID,),
    in_specs=[
        pl.BlockSpec((TB, WORD_DIM), lambda i: (i, 0)),
        pl.BlockSpec((TB, HIDDEN), lambda i: (i, 0)),
        _full((WORD_DIM, HIDDEN)),
        _full((HIDDEN, HIDDEN)), _full((1, HIDDEN)),
        _full((HIDDEN, HIDDEN)), _full((1, HIDDEN)),
        _full((HIDDEN, HIDDEN)), _full((1, HIDDEN)),
        _full((HIDDEN, HIDDEN)), _full((1, HIDDEN)),
    ],
    out_specs=pl.BlockSpec((TB, HIDDEN), lambda i: (i, 0)),
    out_shape=jax.ShapeDtypeStruct((T, HIDDEN), jnp.float32),
)


@jax.jit
def kernel(w_idx, c_idx, word_table, char_table, W_proj,
           Wg0, bg0, Wt0, bt0, Wg1, bg1, Wt1, bt1):
    widx = w_idx.reshape(NW, NWCH, WCH).astype(jnp.int32)
    cp_idx = (c_idx.astype(jnp.int32)
              + jnp.arange(CL, dtype=jnp.int32) * CHAR_VOCAB)
    cidx = cp_idx.reshape(NW, NCIR, 128)
    wpc = W_proj[:, WORD_DIM:].reshape(HIDDEN, CL, CHAR_DIM)
    wpc = jnp.transpose(wpc, (1, 2, 0))
    cp = _cp_call(char_table, wpc)
    word_rows, char_sum = _sc_gather(word_table, cp, widx, cidx)
    out = _tc_call(
        word_rows, char_sum,
        W_proj[:, :WORD_DIM].T,
        Wg0.T, bg0.reshape(1, HIDDEN), Wt0.T, bt0.reshape(1, HIDDEN),
        Wg1.T, bg1.reshape(1, HIDDEN), Wt1.T, bt1.reshape(1, HIDDEN),
    )
    return out.reshape(B, L, HIDDEN)


# speedup vs baseline: 1.1230x; 1.1230x over previous
import functools

import jax
import jax.numpy as jnp
from jax import lax
from jax.experimental import pallas as pl
from jax.experimental.pallas import tpu as pltpu
from jax.experimental.pallas import tpu_sc as plsc

B, L, CL = 1024, 50, 16
WORD_DIM, CHAR_DIM, HIDDEN = 128, 64, 128
CHAR_VOCAB = 1000
T = B * L
NC, NS = 2, 16
NW = NC * NS
TPW = T // NW
WCH = 80
NWCH = TPW // WCH
CT = 16
NCT = TPW // CT
CIW = CT * CL // 128
NCIR = TPW * CL // 128

_sc_mesh = plsc.VectorSubcoreMesh(core_axis_name="c", subcore_axis_name="s")


@functools.partial(
    pl.kernel,
    mesh=_sc_mesh,
    out_type=(
        jax.ShapeDtypeStruct((T, WORD_DIM), jnp.float32),
        jax.ShapeDtypeStruct((T, HIDDEN), jnp.float32),
    ),
    scratch_types=[
        pltpu.VMEM((NWCH, WCH), jnp.int32),
        pltpu.VMEM((NCIR, 128), jnp.int32),
        pltpu.VMEM((2, WCH, WORD_DIM), jnp.float32),
        pltpu.VMEM((2, CT * CL, HIDDEN), jnp.float32),
        pltpu.VMEM((2, CT, HIDDEN), jnp.float32),
        [pltpu.SemaphoreType.DMA] * 2,
        [pltpu.SemaphoreType.DMA] * 2,
    ],
)
def _sc_gather(wt_hbm, cp_hbm, widx_hbm, cidx_hbm, wout_hbm, cout_hbm,
               widx_v, cidx_v, wrows_v, crows_v, csum_v, gsems, osems):
    wid = lax.axis_index("s") * NC + lax.axis_index("c")
    pltpu.sync_copy(widx_hbm.at[wid], widx_v)
    pltpu.sync_copy(cidx_hbm.at[wid], cidx_v)
    base = wid * TPW

    pltpu.async_copy(wt_hbm.at[widx_v.at[0]], wrows_v.at[0], gsems[0])

    def wpair(p, _):
        for b in range(2):
            j = p * 2 + b

            @pl.when(j + 1 < NWCH)
            def _():
                pltpu.async_copy(wt_hbm.at[widx_v.at[j + 1]],
                                 wrows_v.at[1 - b], gsems[1 - b])

            pltpu.make_async_copy(wt_hbm.at[widx_v.at[j]],
                                  wrows_v.at[b], gsems[b]).wait()
            pltpu.sync_copy(wrows_v.at[b],
                            wout_hbm.at[pl.ds(base + j * WCH, WCH)])
        return 0

    lax.fori_loop(0, NWCH // 2, wpair, 0)

    for q in range(CIW):
        pltpu.async_copy(cp_hbm.at[cidx_v.at[q]],
                         crows_v.at[(0, pl.ds(q * 128, 128))], gsems[0])

    def cpair(p, _):
        for b in range(2):
            j = p * 2 + b

            @pl.when(j + 1 < NCT)
            def _():
                for q in range(CIW):
                    pltpu.async_copy(
                        cp_hbm.at[cidx_v.at[(j + 1) * CIW + q]],
                        crows_v.at[(1 - b, pl.ds(q * 128, 128))],
                        gsems[1 - b])

            for q in range(CIW):
                pltpu.make_async_copy(
                    cp_hbm.at[cidx_v.at[j * CIW + q]],
                    crows_v.at[(b, pl.ds(q * 128, 128))], gsems[b]).wait()

            @pl.when(j >= 2)
            def _():
                pltpu.make_async_copy(
                    csum_v.at[b],
                    cout_hbm.at[pl.ds(base + (j - 2) * CT, CT)],
                    osems[b]).wait()

            def tok(i, _):
                for r in range(HIDDEN // 16):
                    sl = pl.ds(r * 16, 16)
                    acc = crows_v[b, i * CL, sl]
                    for k in range(1, CL):
                        acc = acc + crows_v[b, i * CL + k, sl]
                    csum_v[b, i, sl] = acc
                return 0

            lax.fori_loop(0, CT, tok, 0)
            pltpu.async_copy(csum_v.at[b],
                             cout_hbm.at[pl.ds(base + j * CT, CT)],
                             osems[b])
        return 0

    lax.fori_loop(0, NCT // 2, cpair, 0)
    for b in range(2):
        pltpu.make_async_copy(
            csum_v.at[b],
            cout_hbm.at[pl.ds(base + (NCT - 2 + b) * CT, CT)],
            osems[b]).wait()


def _cp_body(ct_ref, wpc_ref, cp_ref):
    ct = ct_ref[...]
    for k in range(CL):
        cp_ref[pl.ds(k * CHAR_VOCAB, CHAR_VOCAB), :] = jnp.dot(
            ct, wpc_ref[k], preferred_element_type=jnp.float32)


_cp_call = pl.pallas_call(
    _cp_body,
    in_specs=[
        pl.BlockSpec((CHAR_VOCAB, CHAR_DIM), lambda: (0, 0)),
        pl.BlockSpec((CL, CHAR_DIM, HIDDEN), lambda: (0, 0, 0)),
    ],
    out_specs=pl.BlockSpec((CL * CHAR_VOCAB, HIDDEN), lambda: (0, 0)),
    out_shape=jax.ShapeDtypeStruct((CL * CHAR_VOCAB, HIDDEN), jnp.float32),
)


TB = 2048
GRID = T // TB


def _tc_body(wd, cs, wpwT, wg0T, bg0, wt0T, bt0, wg1T, bg1, wt1T, bt1, out):
    x = jnp.dot(wd[...], wpwT[...], preferred_element_type=jnp.float32)
    x += cs[...]
    for wgT, bg, wtT, bt in ((wg0T, bg0, wt0T, bt0), (wg1T, bg1, wt1T, bt1)):
        zg = jnp.dot(x, wgT[...], preferred_element_type=jnp.float32) + bg[...]
        g = 1.0 / (1.0 + jnp.exp(-zg))
        zt = jnp.dot(x, wtT[...], preferred_element_type=jnp.float32) + bt[...]
        x = g * jnp.maximum(zt, 0.0) + (1.0 - g) * x
    out[...] = x


def _full(shape):
    return pl.BlockSpec(shape, lambda i: (0, 0))


_tc_call = pl.pallas_call(
    _tc_body,
    grid=(GRID,),
    in_specs=[
        pl.BlockSpec((TB, WORD_DIM), lambda i: (i, 0)),
        pl.BlockSpec((TB, HIDDEN), lambda i: (i, 0)),
        _full((WORD_DIM, HIDDEN)),
        _full((HIDDEN, HIDDEN)), _full((1, HIDDEN)),
        _full((HIDDEN, HIDDEN)), _full((1, HIDDEN)),
        _full((HIDDEN, HIDDEN)), _full((1, HIDDEN)),
        _full((HIDDEN, HIDDEN)), _full((1, HIDDEN)),
    ],
    out_specs=pl.BlockSpec((TB, HIDDEN), lambda i: (i, 0)),
    out_shape=jax.ShapeDtypeStruct((T, HIDDEN), jnp.float32),
)


@jax.jit
def kernel(w_idx, c_idx, word_table, char_table, W_proj,
           Wg0, bg0, Wt0, bt0, Wg1, bg1, Wt1, bt1):
    widx = w_idx.reshape(NW, NWCH, WCH).astype(jnp.int32)
    cp_idx = (c_idx.astype(jnp.int32)
              + jnp.arange(CL, dtype=jnp.int32) * CHAR_VOCAB)
    cidx = cp_idx.reshape(NW, NCIR, 128)
    wpc = W_proj[:, WORD_DIM:].reshape(HIDDEN, CL, CHAR_DIM)
    wpc = jnp.transpose(wpc, (1, 2, 0))
    cp = _cp_call(char_table, wpc)
    word_rows, char_sum = _sc_gather(word_table, cp, widx, cidx)
    out = _tc_call(
        word_rows, char_sum,
        W_proj[:, :WORD_DIM].T,
        Wg0.T, bg0.reshape(1, HIDDEN), Wt0.T, bt0.reshape(1, HIDDEN),
        Wg1.T, bg1.reshape(1, HIDDEN), Wt1.T, bt1.reshape(1, HIDDEN),
    )
    return out.reshape(B, L, HIDDEN)


# speedup vs baseline: 1.2050x; 1.0730x over previous
import functools

import jax
import jax.numpy as jnp
from jax import lax
from jax.experimental import pallas as pl
from jax.experimental.pallas import tpu as pltpu
from jax.experimental.pallas import tpu_sc as plsc

B, L, CL = 1024, 50, 16
WORD_DIM, CHAR_DIM, HIDDEN = 128, 64, 128
CHAR_VOCAB = 1000
T = B * L
NC, NS = 2, 16
NW = NC * NS
TPW = T // NW
WCH = 80
NWCH = TPW // WCH
CT = 16
NCT = TPW // CT
CIW = CT * CL // 128
NCIR = TPW * CL // 128

_sc_mesh = plsc.VectorSubcoreMesh(core_axis_name="c", subcore_axis_name="s")


@functools.partial(
    pl.kernel,
    mesh=_sc_mesh,
    out_type=(
        jax.ShapeDtypeStruct((T, WORD_DIM), jnp.float32),
        jax.ShapeDtypeStruct((T, HIDDEN), jnp.float32),
    ),
    scratch_types=[
        pltpu.VMEM((NWCH, WCH), jnp.int32),
        pltpu.VMEM((NCIR, 128), jnp.int32),
        pltpu.VMEM((2, WCH, WORD_DIM), jnp.float32),
        pltpu.VMEM((2, CT * CL, HIDDEN), jnp.float32),
        pltpu.VMEM((2, CT, HIDDEN), jnp.float32),
        [pltpu.SemaphoreType.DMA] * 2,
        [pltpu.SemaphoreType.DMA] * 2,
    ],
)
def _sc_gather(wt_hbm, cp_hbm, widx_hbm, cidx_hbm, wout_hbm, cout_hbm,
               widx_v, cidx_v, wrows_v, crows_v, csum_v, gsems, osems):
    wid = lax.axis_index("s") * NC + lax.axis_index("c")
    pltpu.sync_copy(widx_hbm.at[wid], widx_v)
    pltpu.sync_copy(cidx_hbm.at[wid], cidx_v)
    base = wid * TPW

    pltpu.async_copy(wt_hbm.at[widx_v.at[0]], wrows_v.at[0], gsems[0])

    def wpair(p, _):
        for b in range(2):
            j = p * 2 + b

            @pl.when(j + 1 < NWCH)
            def _():
                pltpu.async_copy(wt_hbm.at[widx_v.at[j + 1]],
                                 wrows_v.at[1 - b], gsems[1 - b])

            pltpu.make_async_copy(wt_hbm.at[widx_v.at[j]],
                                  wrows_v.at[b], gsems[b]).wait()
            pltpu.sync_copy(wrows_v.at[b],
                            wout_hbm.at[pl.ds(base + j * WCH, WCH)])
        return 0

    lax.fori_loop(0, NWCH // 2, wpair, 0)

    for q in range(CIW):
        pltpu.async_copy(cp_hbm.at[cidx_v.at[q]],
                         crows_v.at[(0, pl.ds(q * 128, 128))], gsems[0])

    def cpair(p, _):
        for b in range(2):
            j = p * 2 + b

            @pl.when(j + 1 < NCT)
            def _():
                for q in range(CIW):
                    pltpu.async_copy(
                        cp_hbm.at[cidx_v.at[(j + 1) * CIW + q]],
                        crows_v.at[(1 - b, pl.ds(q * 128, 128))],
                        gsems[1 - b])

            for q in range(CIW):
                pltpu.make_async_copy(
                    cp_hbm.at[cidx_v.at[j * CIW + q]],
                    crows_v.at[(b, pl.ds(q * 128, 128))], gsems[b]).wait()

            @pl.when(j >= 2)
            def _():
                pltpu.make_async_copy(
                    csum_v.at[b],
                    cout_hbm.at[pl.ds(base + (j - 2) * CT, CT)],
                    osems[b]).wait()

            def tok(i, _):
                for r in range(HIDDEN // 16):
                    sl = pl.ds(r * 16, 16)
                    acc = crows_v[b, i * CL, sl]
                    for k in range(1, CL):
                        acc = acc + crows_v[b, i * CL + k, sl]
                    csum_v[b, i, sl] = acc
                return 0

            lax.fori_loop(0, CT, tok, 0)
            pltpu.async_copy(csum_v.at[b],
                             cout_hbm.at[pl.ds(base + j * CT, CT)],
                             osems[b])
        return 0

    lax.fori_loop(0, NCT // 2, cpair, 0)
    for b in range(2):
        pltpu.make_async_copy(
            csum_v.at[b],
            cout_hbm.at[pl.ds(base + (NCT - 2 + b) * CT, CT)],
            osems[b]).wait()


def _cp_body(ct_ref, wpc_ref, cp_ref):
    ct = ct_ref[...]
    for k in range(CL):
        cp_ref[pl.ds(k * CHAR_VOCAB, CHAR_VOCAB), :] = jnp.dot(
            ct, wpc_ref[k], preferred_element_type=jnp.float32)


_cp_call = pl.pallas_call(
    _cp_body,
    in_specs=[
        pl.BlockSpec((CHAR_VOCAB, CHAR_DIM), lambda: (0, 0)),
        pl.BlockSpec((CL, CHAR_DIM, HIDDEN), lambda: (0, 0, 0)),
    ],
    out_specs=pl.BlockSpec((CL * CHAR_VOCAB, HIDDEN), lambda: (0, 0)),
    out_shape=jax.ShapeDtypeStruct((CL * CHAR_VOCAB, HIDDEN), jnp.float32),
)


BB = 64
TB = BB * L
GRID = B // BB


def _tc_body(wd, cs, wpwT, wg0T, bg0, wt0T, bt0, wg1T, bg1, wt1T, bt1, out):
    x = jnp.dot(wd[...], wpwT[...], preferred_element_type=jnp.float32)
    x += cs[...]
    for wgT, bg, wtT, bt in ((wg0T, bg0, wt0T, bt0), (wg1T, bg1, wt1T, bt1)):
        zg = jnp.dot(x, wgT[...], preferred_element_type=jnp.float32) + bg[...]
        g = 1.0 / (1.0 + jnp.exp(-zg))
        zt = jnp.dot(x, wtT[...], preferred_element_type=jnp.float32) + bt[...]
        x = g * jnp.maximum(zt, 0.0) + (1.0 - g) * x
    out[...] = x.reshape(BB, L, HIDDEN)


def _full(shape):
    return pl.BlockSpec(shape, lambda i: (0, 0))


_tc_call = pl.pallas_call(
    _tc_body,
    grid=(GRID,),
    in_specs=[
        pl.BlockSpec((TB, WORD_DIM), lambda i: (i, 0)),
        pl.BlockSpec((TB, HIDDEN), lambda i: (i, 0)),
        _full((WORD_DIM, HIDDEN)),
        _full((HIDDEN, HIDDEN)), _full((1, HIDDEN)),
        _full((HIDDEN, HIDDEN)), _full((1, HIDDEN)),
        _full((HIDDEN, HIDDEN)), _full((1, HIDDEN)),
        _full((HIDDEN, HIDDEN)), _full((1, HIDDEN)),
    ],
    out_specs=pl.BlockSpec((BB, L, HIDDEN), lambda i: (i, 0, 0)),
    out_shape=jax.ShapeDtypeStruct((B, L, HIDDEN), jnp.float32),
)


@jax.jit
def kernel(w_idx, c_idx, word_table, char_table, W_proj,
           Wg0, bg0, Wt0, bt0, Wg1, bg1, Wt1, bt1):
    widx = w_idx.reshape(NW, NWCH, WCH).astype(jnp.int32)
    cidx = (c_idx.astype(jnp.int32).reshape(NW, NCIR, 128)
            + (jnp.arange(128, dtype=jnp.int32) % CL) * CHAR_VOCAB)
    wpc = W_proj[:, WORD_DIM:].reshape(HIDDEN, CL, CHAR_DIM)
    wpc = jnp.transpose(wpc, (1, 2, 0))
    cp = _cp_call(char_table, wpc)
    word_rows, char_sum = _sc_gather(word_table, cp, widx, cidx)
    out = _tc_call(
        word_rows, char_sum,
        W_proj[:, :WORD_DIM].T,
        Wg0.T, bg0.reshape(1, HIDDEN), Wt0.T, bt0.reshape(1, HIDDEN),
        Wg1.T, bg1.reshape(1, HIDDEN), Wt1.T, bt1.reshape(1, HIDDEN),
    )
    return out
